# baseline (speedup 1.0000x reference)
import functools

import jax
import jax.numpy as jnp
from jax.experimental import pallas as pl
from jax.experimental.pallas import tpu as pltpu


def _se_fused_kernel(x_ref, m_ref, w1e_ref, w1o_ref, b1_ref,
                     w2e_ref, w2o_ref, b2e_ref, b2o_ref, o_ref,
                     *, hw, inv_hw):
    x = x_ref[0].astype(jnp.float32)
    c2, l = x.shape
    s2 = jax.lax.dot_general(x, m_ref[...],
                             (((1,), (1,)), ((), ())),
                             preferred_element_type=jnp.float32)
    pe = s2[:, 0:1] * inv_hw
    po = s2[:, 1:2] * inv_hw
    h = (jnp.dot(w1e_ref[...], pe, preferred_element_type=jnp.float32)
         + jnp.dot(w1o_ref[...], po, preferred_element_type=jnp.float32))
    h = jnp.maximum(h + b1_ref[...], 0.0)
    ge = jax.nn.sigmoid(
        jnp.dot(w2e_ref[...], h, preferred_element_type=jnp.float32)
        + b2e_ref[...])
    go = jax.nn.sigmoid(
        jnp.dot(w2o_ref[...], h, preferred_element_type=jnp.float32)
        + b2o_ref[...])
    pos = jax.lax.broadcasted_iota(jnp.int32, (c2, l), 1)
    gate = jnp.where(pos < hw, ge, go)
    o_ref[0] = (x * gate).astype(o_ref.dtype)


def kernel(x, w_reduce, b_reduce, w_expand, b_expand):
    N, C, H, W = x.shape
    hw = H * W
    cr = w_reduce.shape[0]
    c2 = C // 2
    l = 2 * hw

    xs = x.reshape(N, c2, l)
    half = (jnp.arange(l, dtype=jnp.float32) < hw).astype(jnp.float32)
    mask = jnp.stack([half, 1.0 - half])

    w1 = w_reduce.astype(jnp.float32)
    b1 = b_reduce.astype(jnp.float32)
    w2 = w_expand.astype(jnp.float32)
    b2 = b_expand.astype(jnp.float32)
    w1e, w1o = w1[:, 0::2], w1[:, 1::2]
    w2e, w2o = w2[0::2, :], w2[1::2, :]
    b2e, b2o = b2[0::2, :], b2[1::2, :]

    y = pl.pallas_call(
        functools.partial(_se_fused_kernel, hw=hw, inv_hw=1.0 / float(hw)),
        out_shape=jax.ShapeDtypeStruct((N, c2, l), x.dtype),
        grid=(N,),
        in_specs=[
            pl.BlockSpec((1, c2, l), lambda n: (n, 0, 0)),
            pl.BlockSpec((2, l), lambda n: (0, 0)),
            pl.BlockSpec((cr, c2), lambda n: (0, 0)),
            pl.BlockSpec((cr, c2), lambda n: (0, 0)),
            pl.BlockSpec((cr, 1), lambda n: (0, 0)),
            pl.BlockSpec((c2, cr), lambda n: (0, 0)),
            pl.BlockSpec((c2, cr), lambda n: (0, 0)),
            pl.BlockSpec((c2, 1), lambda n: (0, 0)),
            pl.BlockSpec((c2, 1), lambda n: (0, 0)),
        ],
        out_specs=pl.BlockSpec((1, c2, l), lambda n: (n, 0, 0)),
        compiler_params=pltpu.CompilerParams(
            dimension_semantics=("parallel",)),
        cost_estimate=pl.CostEstimate(
            flops=int(2 * N * C * hw + 4 * N * C * cr + 2 * N * C * hw),
            transcendentals=int(N * C),
            bytes_accessed=int(2 * xs.size * x.dtype.itemsize
                               + (w1.size + b1.size + w2.size + b2.size) * 4),
        ),
    )(xs, mask, w1e, w1o, b1, w2e, w2o, b2e, b2o)

    return y.reshape(N, C, H, W)


# speedup vs baseline: 2.5262x; 2.5262x over previous
import functools

import jax
import jax.numpy as jnp
from jax.experimental import pallas as pl
from jax.experimental.pallas import tpu as pltpu

_NSLOT = 6
_PREF = 3
_NCHUNK = 2


def _se_kernel(x_hbm, w1_ref, b1_ref, w2_ref, b2_ref, y_hbm,
               xbuf, in_sem, out_sem, *, inv_hw):
    n_b, c, hw = x_hbm.shape
    cq = c // _NCHUNK

    def start_in(n, slot):
        for q in range(_NCHUNK):
            pltpu.make_async_copy(
                x_hbm.at[n, pl.ds(q * cq, cq)],
                xbuf.at[slot, pl.ds(q * cq, cq)],
                in_sem.at[slot, q]).start()

    def wait_in(slot):
        for q in range(_NCHUNK):
            pltpu.make_async_copy(
                x_hbm.at[0, pl.ds(q * cq, cq)],
                xbuf.at[slot, pl.ds(q * cq, cq)],
                in_sem.at[slot, q]).wait()

    def start_out(n, slot):
        for q in range(_NCHUNK):
            pltpu.make_async_copy(
                xbuf.at[slot, pl.ds(q * cq, cq)],
                y_hbm.at[n, pl.ds(q * cq, cq)],
                out_sem.at[slot, q]).start()

    def wait_out(slot):
        for q in range(_NCHUNK):
            pltpu.make_async_copy(
                xbuf.at[slot, pl.ds(q * cq, cq)],
                y_hbm.at[0, pl.ds(q * cq, cq)],
                out_sem.at[slot, q]).wait()

    for n in range(_PREF):
        start_in(n, n % _NSLOT)

    def body(n, _):
        slot = jax.lax.rem(n, _NSLOT)

        @pl.when(n + _PREF < n_b)
        def _():
            tgt = jax.lax.rem(n + _PREF, _NSLOT)

            @pl.when(n + _PREF >= _NSLOT)
            def _():
                wait_out(tgt)
            start_in(n + _PREF, tgt)

        wait_in(slot)
        x = xbuf[slot]
        pooled = jnp.sum(x, axis=-1, keepdims=True) * inv_hw
        h = jnp.dot(w1_ref[...], pooled,
                    preferred_element_type=jnp.float32)
        h = jnp.maximum(h + b1_ref[...], 0.0)
        z = jnp.dot(w2_ref[...], h,
                    preferred_element_type=jnp.float32)
        g = jax.nn.sigmoid(z + b2_ref[...])
        xbuf[slot] = x * g
        start_out(n, slot)
        return ()

    jax.lax.fori_loop(0, n_b, body, (), unroll=False)

    for k in range(min(_NSLOT, n_b)):
        wait_out((n_b - 1 - k) % _NSLOT)


def kernel(x, w_reduce, b_reduce, w_expand, b_expand):
    N, C, H, W = x.shape
    hw = H * W
    cr = w_reduce.shape[0]

    xf = x.reshape(N, C, hw)
    w1 = w_reduce.astype(jnp.float32)
    b1 = b_reduce.astype(jnp.float32)
    w2 = w_expand.astype(jnp.float32)
    b2 = b_expand.astype(jnp.float32)

    y = pl.pallas_call(
        functools.partial(_se_kernel, inv_hw=1.0 / float(hw)),
        out_shape=jax.ShapeDtypeStruct((N, C, hw), x.dtype),
        in_specs=[
            pl.BlockSpec(memory_space=pltpu.MemorySpace.HBM),
            pl.BlockSpec((cr, C), lambda: (0, 0)),
            pl.BlockSpec((cr, 1), lambda: (0, 0)),
            pl.BlockSpec((C, cr), lambda: (0, 0)),
            pl.BlockSpec((C, 1), lambda: (0, 0)),
        ],
        out_specs=pl.BlockSpec(memory_space=pltpu.MemorySpace.HBM),
        scratch_shapes=[
            pltpu.VMEM((_NSLOT, C, hw), jnp.float32),
            pltpu.SemaphoreType.DMA((_NSLOT, _NCHUNK)),
            pltpu.SemaphoreType.DMA((_NSLOT, _NCHUNK)),
        ],
        cost_estimate=pl.CostEstimate(
            flops=int(2 * N * C * hw + 4 * N * C * cr),
            transcendentals=int(N * C),
            bytes_accessed=int(2 * xf.size * x.dtype.itemsize
                               + (w1.size + b1.size + w2.size + b2.size) * 4),
        ),
    )(xf, w1, b1, w2, b2)

    return y.reshape(N, C, H, W)


# speedup vs baseline: 2.5267x; 1.0002x over previous
import functools

import jax
import jax.numpy as jnp
from jax.experimental import pallas as pl
from jax.experimental.pallas import tpu as pltpu

_NSLOT = 6
_PREF = 3
_NCHUNK = 2


def _se_kernel(x_hbm, w1_ref, b1_ref, w2_ref, b2_ref, y_hbm,
               xbuf, in_sem, out_sem, *, inv_hw):
    n_b, c, hw = x_hbm.shape
    cq = c // _NCHUNK

    def start_in(n, slot):
        for q in range(_NCHUNK):
            pltpu.make_async_copy(
                x_hbm.at[n, pl.ds(q * cq, cq)],
                xbuf.at[slot, pl.ds(q * cq, cq)],
                in_sem.at[slot, q]).start(priority=0)

    def wait_in(slot):
        for q in range(_NCHUNK):
            pltpu.make_async_copy(
                x_hbm.at[0, pl.ds(q * cq, cq)],
                xbuf.at[slot, pl.ds(q * cq, cq)],
                in_sem.at[slot, q]).wait()

    def start_out(n, slot):
        for q in range(_NCHUNK):
            pltpu.make_async_copy(
                xbuf.at[slot, pl.ds(q * cq, cq)],
                y_hbm.at[n, pl.ds(q * cq, cq)],
                out_sem.at[slot, q]).start(priority=1)

    def wait_out(slot):
        for q in range(_NCHUNK):
            pltpu.make_async_copy(
                xbuf.at[slot, pl.ds(q * cq, cq)],
                y_hbm.at[0, pl.ds(q * cq, cq)],
                out_sem.at[slot, q]).wait()

    for n in range(_PREF):
        start_in(n, n % _NSLOT)

    def body(n, _):
        slot = jax.lax.rem(n, _NSLOT)

        @pl.when(n + _PREF < n_b)
        def _():
            tgt = jax.lax.rem(n + _PREF, _NSLOT)

            @pl.when(n + _PREF >= _NSLOT)
            def _():
                wait_out(tgt)
            start_in(n + _PREF, tgt)

        wait_in(slot)
        x = xbuf[slot]
        pooled = jnp.sum(x, axis=-1, keepdims=True) * inv_hw
        h = jnp.dot(w1_ref[...], pooled,
                    preferred_element_type=jnp.float32)
        h = jnp.maximum(h + b1_ref[...], 0.0)
        z = jnp.dot(w2_ref[...], h,
                    preferred_element_type=jnp.float32)
        g = jax.nn.sigmoid(z + b2_ref[...])
        xbuf[slot] = x * g
        start_out(n, slot)
        return ()

    jax.lax.fori_loop(0, n_b, body, (), unroll=False)

    for k in range(min(_NSLOT, n_b)):
        wait_out((n_b - 1 - k) % _NSLOT)


def kernel(x, w_reduce, b_reduce, w_expand, b_expand):
    N, C, H, W = x.shape
    hw = H * W
    cr = w_reduce.shape[0]

    xf = x.reshape(N, C, hw)
    w1 = w_reduce.astype(jnp.float32)
    b1 = b_reduce.astype(jnp.float32)
    w2 = w_expand.astype(jnp.float32)
    b2 = b_expand.astype(jnp.float32)

    y = pl.pallas_call(
        functools.partial(_se_kernel, inv_hw=1.0 / float(hw)),
        out_shape=jax.ShapeDtypeStruct((N, C, hw), x.dtype),
        in_specs=[
            pl.BlockSpec(memory_space=pltpu.MemorySpace.HBM),
            pl.BlockSpec((cr, C), lambda: (0, 0)),
            pl.BlockSpec((cr, 1), lambda: (0, 0)),
            pl.BlockSpec((C, cr), lambda: (0, 0)),
            pl.BlockSpec((C, 1), lambda: (0, 0)),
        ],
        out_specs=pl.BlockSpec(memory_space=pltpu.MemorySpace.HBM),
        scratch_shapes=[
            pltpu.VMEM((_NSLOT, C, hw), jnp.float32),
            pltpu.SemaphoreType.DMA((_NSLOT, _NCHUNK)),
            pltpu.SemaphoreType.DMA((_NSLOT, _NCHUNK)),
        ],
        cost_estimate=pl.CostEstimate(
            flops=int(2 * N * C * hw + 4 * N * C * cr),
            transcendentals=int(N * C),
            bytes_accessed=int(2 * xf.size * x.dtype.itemsize
                               + (w1.size + b1.size + w2.size + b2.size) * 4),
        ),
    )(xf, w1, b1, w2, b2)

    return y.reshape(N, C, H, W)
